# Initial kernel scaffold; baseline (speedup 1.0000x reference)
import jax
import jax.numpy as jnp
from jax import lax
from jax.experimental import pallas as pl
from jax.experimental.pallas import tpu as pltpu
from jax.experimental.pallas import tpu_sc as plsc

N_ROWS = 320000
N_COLS = 128
N_GROUPS = 10000
ACC_ROWS = 10240
CHUNK = 128
N_CHUNKS = N_ROWS // CHUNK
NC, NS = 2, 16
NW = NC * NS
ITERS = (N_CHUNKS + NW - 1) // NW
STRIPE = ACC_ROWS // NS


def _sc_body(x_hbm, grp_hbm, zx_hbm, z16_hbm, o16_hbm, psum_hbm, pcnt_hbm,
             xbuf, cbuf, idxbuf, psum_acc, pcnt_acc):
    cid = lax.axis_index("c")
    sid = lax.axis_index("s")
    wid = sid * NC + cid

    pltpu.sync_copy(zx_hbm, xbuf)
    pltpu.sync_copy(z16_hbm, cbuf)
    for b in range(STRIPE // CHUNK):
        r0 = sid * STRIPE + b * CHUNK
        pltpu.sync_copy(xbuf, psum_acc.at[pl.ds(r0, CHUNK)])
        pltpu.sync_copy(cbuf, pcnt_acc.at[pl.ds(r0, CHUNK)])
    pltpu.sync_copy(o16_hbm, cbuf)
    plsc.subcore_barrier()

    def body(i, carry):
        c = wid + NW * i

        @pl.when(c < N_CHUNKS)
        def _():
            pltpu.sync_copy(grp_hbm.at[pl.ds(c, 1)], idxbuf)
            pltpu.sync_copy(x_hbm.at[pl.ds(c * CHUNK, CHUNK)], xbuf)
            pltpu.sync_copy(xbuf, psum_acc.at[idxbuf.at[0]], add=True)
            pltpu.sync_copy(cbuf, pcnt_acc.at[idxbuf.at[0]], add=True)

        return carry

    lax.fori_loop(0, ITERS, body, 0)
    plsc.subcore_barrier()

    for b in range(STRIPE // CHUNK):
        r0 = sid * STRIPE + b * CHUNK
        pltpu.sync_copy(psum_acc.at[pl.ds(r0, CHUNK)], xbuf)
        pltpu.sync_copy(xbuf, psum_hbm.at[cid, pl.ds(r0, CHUNK)])
        pltpu.sync_copy(pcnt_acc.at[pl.ds(r0, CHUNK)], cbuf)
        pltpu.sync_copy(cbuf, pcnt_hbm.at[cid, pl.ds(r0, CHUNK)])


def _merge_body(ps_ref, pc_ref, out_ref):
    s = ps_ref[0] + ps_ref[1]
    c = pc_ref[0][:, 0:1] + pc_ref[1][:, 0:1]
    out_ref[...] = s / c


def kernel(x, group):
    grp = group.astype(jnp.int32).reshape(N_CHUNKS, CHUNK)
    zx = jnp.zeros((CHUNK, N_COLS), jnp.float32)
    z16 = jnp.zeros((CHUNK, 16), jnp.float32)
    o16 = jnp.ones((CHUNK, 16), jnp.float32)

    sc = pl.kernel(
        _sc_body,
        out_type=(
            jax.ShapeDtypeStruct((NC, ACC_ROWS, N_COLS), jnp.float32),
            jax.ShapeDtypeStruct((NC, ACC_ROWS, 16), jnp.float32),
        ),
        mesh=plsc.VectorSubcoreMesh(core_axis_name="c", subcore_axis_name="s"),
        scratch_types=[
            pltpu.VMEM((CHUNK, N_COLS), jnp.float32),
            pltpu.VMEM((CHUNK, 16), jnp.float32),
            pltpu.VMEM((1, CHUNK), jnp.int32),
            pltpu.VMEM_SHARED((ACC_ROWS, N_COLS), jnp.float32),
            pltpu.VMEM_SHARED((ACC_ROWS, 16), jnp.float32),
        ],
    )
    psum, pcnt = sc(x, grp, zx, z16, o16)

    nblk = 10
    blk = N_GROUPS // nblk
    out = pl.pallas_call(
        _merge_body,
        grid=(nblk,),
        in_specs=[
            pl.BlockSpec((NC, blk, N_COLS), lambda i: (0, i, 0)),
            pl.BlockSpec((NC, blk, 16), lambda i: (0, i, 0)),
        ],
        out_specs=pl.BlockSpec((blk, N_COLS), lambda i: (i, 0)),
        out_shape=jax.ShapeDtypeStruct((N_GROUPS, N_COLS), jnp.float32),
    )(psum, pcnt)
    return out


# speedup vs baseline: 8.8636x; 8.8636x over previous
import jax
import jax.numpy as jnp
from jax import lax
from jax.experimental import pallas as pl
from jax.experimental.pallas import tpu as pltpu
from jax.experimental.pallas import tpu_sc as plsc

N_ROWS = 320000
N_COLS = 128
N_GROUPS = 10000
HALF_G = N_GROUPS // 2
ACC_H = 5120
CHUNK = 128
N_CHUNKS = N_ROWS // CHUNK
NC, NS = 2, 16
FULL_ITERS = N_CHUNKS // NS
STRIPE = ACC_H // NS
L = 16


def _sc_body(x_hbm, grp_hbm, zx_hbm, ones_hbm, psum_hbm, pcnt_hbm,
             xbuf, onesbuf, idxbuf, idx2buf, acc, cnt):
    cid = lax.axis_index("c")
    sid = lax.axis_index("s")
    lo = cid * HALF_G
    hi = lo + HALF_G

    pltpu.sync_copy(zx_hbm, xbuf)
    for off, n in ((0, 128), (128, 128), (256, 64)):
        r0 = sid * STRIPE + off
        pltpu.sync_copy(xbuf.at[pl.ds(0, n)], acc.at[pl.ds(r0, n)])
        pltpu.sync_copy(xbuf.at[pl.ds(0, n)], cnt.at[pl.ds(r0, n)])
    pltpu.sync_copy(ones_hbm, onesbuf)
    plsc.subcore_barrier()

    def step(c):
        r = c * CHUNK
        pltpu.sync_copy(grp_hbm.at[pl.ds(r, CHUNK)], idxbuf)
        cmin = idxbuf[pl.ds(0, L)][0]
        cmax = idxbuf[pl.ds(CHUNK - L, L)][L - 1]

        @pl.when(jnp.logical_and(cmax >= lo, cmin < hi))
        def _():
            pltpu.sync_copy(x_hbm.at[pl.ds(r, CHUNK)], xbuf)
            for j in range(CHUNK // L):
                v = idxbuf[pl.ds(j * L, L)]
                t = v - lo
                in_range = jnp.logical_and(v >= lo, v < hi)
                idx2buf[pl.ds(j * L, L)] = jnp.where(in_range, t, HALF_G)
            pltpu.sync_copy(xbuf, acc.at[idx2buf], add=True)
            pltpu.sync_copy(onesbuf, cnt.at[idx2buf], add=True)

    def body(i, carry):
        step(sid + NS * i)
        return carry

    lax.fori_loop(0, FULL_ITERS, body, 0)

    @pl.when(sid + NS * FULL_ITERS < N_CHUNKS)
    def _():
        step(sid + NS * FULL_ITERS)

    plsc.subcore_barrier()

    for off, n in ((0, 128), (128, 128), (256, 64)):
        r0 = sid * STRIPE + off
        pltpu.sync_copy(acc.at[pl.ds(r0, n)], xbuf.at[pl.ds(0, n)])
        pltpu.sync_copy(xbuf.at[pl.ds(0, n)],
                        psum_hbm.at[pl.ds(cid * ACC_H + r0, n)])
        pltpu.sync_copy(cnt.at[pl.ds(r0, n)], onesbuf.at[pl.ds(0, n)])
        pltpu.sync_copy(onesbuf.at[pl.ds(0, n)],
                        pcnt_hbm.at[pl.ds(cid * ACC_H + r0, n)])


def _merge_body(ps_ref, pc_ref, out_ref):
    out_ref[...] = ps_ref[0] / pc_ref[0]


def kernel(x, group):
    grp = group.astype(jnp.int32)
    zx = jnp.zeros((CHUNK, N_COLS), jnp.float32)
    ones = jnp.ones((CHUNK, N_COLS), jnp.float32)

    sc = pl.kernel(
        _sc_body,
        out_type=(
            jax.ShapeDtypeStruct((NC * ACC_H, N_COLS), jnp.float32),
            jax.ShapeDtypeStruct((NC * ACC_H, N_COLS), jnp.float32),
        ),
        mesh=plsc.VectorSubcoreMesh(core_axis_name="c", subcore_axis_name="s"),
        scratch_types=[
            pltpu.VMEM((CHUNK, N_COLS), jnp.float32),
            pltpu.VMEM((CHUNK, N_COLS), jnp.float32),
            pltpu.VMEM((CHUNK,), jnp.int32),
            pltpu.VMEM((CHUNK,), jnp.int32),
            pltpu.VMEM_SHARED((ACC_H, N_COLS), jnp.float32),
            pltpu.VMEM_SHARED((ACC_H, N_COLS), jnp.float32),
        ],
    )
    psum, pcnt = sc(x, grp, zx, ones)
    psum = psum.reshape(NC, ACC_H, N_COLS)
    pcnt = pcnt.reshape(NC, ACC_H, N_COLS)

    nblk = 10
    blk = N_GROUPS // nblk
    out = pl.pallas_call(
        _merge_body,
        grid=(nblk,),
        in_specs=[
            pl.BlockSpec((1, blk, N_COLS), lambda i: (i // 5, i % 5, 0)),
            pl.BlockSpec((1, blk, N_COLS), lambda i: (i // 5, i % 5, 0)),
        ],
        out_specs=pl.BlockSpec((blk, N_COLS), lambda i: (i, 0)),
        out_shape=jax.ShapeDtypeStruct((N_GROUPS, N_COLS), jnp.float32),
    )(psum, pcnt)
    return out


# speedup vs baseline: 13.0304x; 1.4701x over previous
import jax
import jax.numpy as jnp
from jax import lax
from jax.experimental import pallas as pl
from jax.experimental.pallas import tpu as pltpu
from jax.experimental.pallas import tpu_sc as plsc

N_ROWS = 320000
N_COLS = 128
N_GROUPS = 10000
HALF_G = N_GROUPS // 2
ACC_H = 5056
CHUNK = 128
N_CHUNKS = N_ROWS // CHUNK
NC, NS = 2, 16
STRIPE = 320
L = 16
SEARCH_STEPS = 12


def _sc_body(x_hbm, grp_hbm, zx_hbm, ones_hbm, psum_hbm, pcnt_hbm,
             xb0, xb1, onesbuf, ib0, ib1, i2b0, i2b1, acc, cnt,
             semx0, semx1, sems0, semc0, sems1, semc1):
    cid = lax.axis_index("c")
    sid = lax.axis_index("s")
    lo = cid * HALF_G
    hi = lo + HALF_G

    pltpu.sync_copy(zx_hbm, xb0)
    base = jnp.minimum(sid * STRIPE, ACC_H - STRIPE)
    for off, n in ((0, 128), (128, 128), (256, 64)):
        r0 = base + off
        pltpu.sync_copy(xb0.at[pl.ds(0, n)], acc.at[pl.ds(r0, n)])
        pltpu.sync_copy(xb0.at[pl.ds(0, n)], cnt.at[pl.ds(r0, n)])
    pltpu.sync_copy(ones_hbm, onesbuf)

    def sbody(_, carry):
        s_lo, s_hi = carry
        mid = lax.div(s_lo + s_hi, 2)
        pltpu.sync_copy(grp_hbm.at[pl.ds(mid * CHUNK, L)], ib0.at[pl.ds(0, L)])
        v = ib0[pl.ds(0, L)][0]
        p = v >= HALF_G
        return jnp.where(p, s_lo, mid + 1), jnp.where(p, mid, s_hi)

    _, cb = lax.fori_loop(0, SEARCH_STEPS, sbody, (0, N_CHUNKS))
    start = jnp.where(cid == 0, 0, jnp.maximum(cb - 1, 0))
    end = jnp.where(cid == 0, cb, N_CHUNKS)
    start_c = start + sid
    n = jnp.maximum(0, lax.div(end - start_c + (NS - 1), NS))

    plsc.subcore_barrier()

    def cofs(i):
        return (start_c + NS * i) * CHUNK

    def cofs_clamped(i):
        return (start_c + NS * jnp.minimum(i, n - 1)) * CHUNK

    def remap(ib, i2b):
        for j in range(CHUNK // L):
            v = ib[pl.ds(j * L, L)]
            in_range = jnp.logical_and(v >= lo, v < hi)
            i2b[pl.ds(j * L, L)] = jnp.where(in_range, v - lo, HALF_G)

    def scatter(xb, i2b, ss, sc_):
        hs = pltpu.async_copy(xb, acc.at[i2b], ss, add=True)
        hc = pltpu.async_copy(onesbuf, cnt.at[i2b], sc_, add=True)
        return hs, hc

    def wait_gather(xb, sem):
        pltpu.make_async_copy(x_hbm.at[pl.ds(0, CHUNK)], xb, sem).wait()

    @pl.when(n > 0)
    def _():
        r0 = cofs(0)
        r1 = cofs_clamped(1)
        pltpu.sync_copy(grp_hbm.at[pl.ds(r0, CHUNK)], ib0)
        pltpu.sync_copy(grp_hbm.at[pl.ds(r1, CHUNK)], ib1)
        pltpu.async_copy(x_hbm.at[pl.ds(r0, CHUNK)], xb0, semx0)
        pltpu.async_copy(x_hbm.at[pl.ds(r1, CHUNK)], xb1, semx1)

        def body(k, carry):
            remap(ib0, i2b0)
            remap(ib1, i2b1)
            wait_gather(xb0, semx0)
            h0 = scatter(xb0, i2b0, sems0, semc0)
            wait_gather(xb1, semx1)
            h1 = scatter(xb1, i2b1, sems1, semc1)
            ra = cofs_clamped(2 * k + 2)
            rb = cofs_clamped(2 * k + 3)
            pltpu.sync_copy(grp_hbm.at[pl.ds(ra, CHUNK)], ib0)
            pltpu.sync_copy(grp_hbm.at[pl.ds(rb, CHUNK)], ib1)
            h0[0].wait()
            h0[1].wait()
            pltpu.async_copy(x_hbm.at[pl.ds(ra, CHUNK)], xb0, semx0)
            h1[0].wait()
            h1[1].wait()
            pltpu.async_copy(x_hbm.at[pl.ds(rb, CHUNK)], xb1, semx1)
            return carry

        lax.fori_loop(0, lax.div(n, 2), body, 0)

        wait_gather(xb0, semx0)
        wait_gather(xb1, semx1)

        @pl.when(lax.rem(n, 2) == 1)
        def _():
            remap(ib0, i2b0)
            h = scatter(xb0, i2b0, sems0, semc0)
            h[0].wait()
            h[1].wait()

    plsc.subcore_barrier()

    for off, n2 in ((0, 128), (128, 128), (256, 64)):
        r0 = jnp.minimum(sid * STRIPE, ACC_H - STRIPE) + off
        pltpu.sync_copy(acc.at[pl.ds(r0, n2)], xb0.at[pl.ds(0, n2)])
        pltpu.sync_copy(xb0.at[pl.ds(0, n2)],
                        psum_hbm.at[pl.ds(cid * ACC_H + r0, n2)])
        pltpu.sync_copy(cnt.at[pl.ds(r0, n2)], xb1.at[pl.ds(0, n2)])
        pltpu.sync_copy(xb1.at[pl.ds(0, n2)],
                        pcnt_hbm.at[pl.ds(cid * ACC_H + r0, n2)])


def _merge_body(ps_ref, pc_ref, out_ref):
    out_ref[...] = ps_ref[0] / pc_ref[0]


def kernel(x, group):
    grp = group.astype(jnp.int32)
    zx = jnp.zeros((CHUNK, N_COLS), jnp.float32)
    ones = jnp.ones((CHUNK, N_COLS), jnp.float32)

    sc = pl.kernel(
        _sc_body,
        out_type=(
            jax.ShapeDtypeStruct((NC * ACC_H, N_COLS), jnp.float32),
            jax.ShapeDtypeStruct((NC * ACC_H, N_COLS), jnp.float32),
        ),
        mesh=plsc.VectorSubcoreMesh(core_axis_name="c", subcore_axis_name="s"),
        scratch_types=[
            pltpu.VMEM((CHUNK, N_COLS), jnp.float32),
            pltpu.VMEM((CHUNK, N_COLS), jnp.float32),
            pltpu.VMEM((CHUNK, N_COLS), jnp.float32),
            pltpu.VMEM((CHUNK,), jnp.int32),
            pltpu.VMEM((CHUNK,), jnp.int32),
            pltpu.VMEM((CHUNK,), jnp.int32),
            pltpu.VMEM((CHUNK,), jnp.int32),
            pltpu.VMEM_SHARED((ACC_H, N_COLS), jnp.float32),
            pltpu.VMEM_SHARED((ACC_H, N_COLS), jnp.float32),
            pltpu.SemaphoreType.DMA,
            pltpu.SemaphoreType.DMA,
            pltpu.SemaphoreType.DMA,
            pltpu.SemaphoreType.DMA,
            pltpu.SemaphoreType.DMA,
            pltpu.SemaphoreType.DMA,
        ],
    )
    psum, pcnt = sc(x, grp, zx, ones)
    psum = psum.reshape(NC, ACC_H, N_COLS)
    pcnt = pcnt.reshape(NC, ACC_H, N_COLS)

    nblk = 10
    blk = N_GROUPS // nblk
    out = pl.pallas_call(
        _merge_body,
        grid=(nblk,),
        in_specs=[
            pl.BlockSpec((1, blk, N_COLS), lambda i: (i // 5, i % 5, 0)),
            pl.BlockSpec((1, blk, N_COLS), lambda i: (i // 5, i % 5, 0)),
        ],
        out_specs=pl.BlockSpec((blk, N_COLS), lambda i: (i, 0)),
        out_shape=jax.ShapeDtypeStruct((N_GROUPS, N_COLS), jnp.float32),
    )(psum, pcnt)
    return out
